# Initial kernel scaffold; baseline (speedup 1.0000x reference)
import jax
import jax.numpy as jnp
from jax.experimental import pallas as pl


def kernel(a, X1, X2, W):
    raise NotImplementedError("write your pallas kernel here")


# speedup vs baseline: 107.8862x; 107.8862x over previous
import functools

import jax
import jax.numpy as jnp
from jax import lax
from jax.experimental import pallas as pl
from jax.experimental.pallas import tpu as pltpu
from jax.experimental.pallas import tpu_sc as plsc

N = 50000
M = 16
WL = 8
PW = 2 * WL
NB = 80
NBLOCKS = N // NB
NC, NS, L = 2, 16, 16
NW = NC * NS
NG = NB // L


def _lane_reduce(v, op):
    iota = lax.iota(jnp.int32, L)
    for d in (8, 4, 2, 1):
        v = op(v, v.at[iota ^ d].get(mode="promise_in_bounds",
                                     unique_indices=True))
    return v


def _f_groups(xb, a_v, m_vec, col_g):
    acc = None
    for w in range(WL):
        i1 = plsc.load_gather(xb, [m_vec, col_g + (2 * w)])
        i2 = plsc.load_gather(xb, [m_vec, col_g + (2 * w + 1)])
        y1 = plsc.load_gather(a_v, [i1])
        y2 = plsc.load_gather(a_v, [i2])
        z = y1 * y2
        acc = z if acc is None else jnp.maximum(acc, z)
    return acc


def _compute_f(xb, a_v, f_v, colbase):
    def mbody(m, carry):
        m_vec = jnp.broadcast_to(m, (L,)).astype(jnp.int32)
        for g in range(NG):
            col_g = colbase + g * (L * PW)
            f_v[m, pl.ds(g * L, L)] = _f_groups(xb, a_v, m_vec, col_g)
        return carry
    lax.fori_loop(0, M, mbody, 0, unroll=False)


def _tec_body(a_hbm, x1_hbm, x2_hbm, w_hbm, out_hbm,
              a_v, xb, f1_v, f2_v, pi_v, out_v):
    cid = lax.axis_index("c")
    sid = lax.axis_index("s")
    wid = sid * NC + cid

    pltpu.sync_copy(a_hbm, a_v)
    pltpu.sync_copy(w_hbm, pi_v)

    rows = [pi_v[i, :] for i in range(M)]
    mx = _lane_reduce(functools.reduce(jnp.maximum, rows), jnp.maximum)
    mxs = mx[0]
    es = [jnp.exp(r - mxs) for r in rows]
    tot = functools.reduce(lambda x, y: x + y, es)
    inv_v = 1.0 / _lane_reduce(tot, lambda x, y: x + y)
    pis = [e * inv_v for e in es]
    pi1 = [_lane_reduce(p, lambda x, y: x + y)[0] for p in pis]
    pi2v = tot * inv_v
    pi2 = [pi2v[m2] for m2 in range(M)]
    pi_s = [[pis[m1][m2] for m2 in range(M)] for m1 in range(M)]

    iota = lax.iota(jnp.int32, L)
    colbase = iota * PW

    num_j = (NBLOCKS - wid + NW - 1) // NW

    def blk(j, carry):
        b = wid + j * NW
        n0 = b * NB

        pltpu.sync_copy(x1_hbm.at[:, pl.ds(n0 * PW, NB * PW)], xb)
        _compute_f(xb, a_v, f1_v, colbase)
        pltpu.sync_copy(x2_hbm.at[:, pl.ds(n0 * PW, NB * PW)], xb)
        _compute_f(xb, a_v, f2_v, colbase)

        for g in range(NG):
            f1g = [f1_v[m, pl.ds(g * L, L)] for m in range(M)]
            f2g = [f2_v[m, pl.ds(g * L, L)] for m in range(M)]
            eu = functools.reduce(
                lambda x, y: x + y, [pi1[m] * f1g[m] for m in range(M)])
            ev = functools.reduce(
                lambda x, y: x + y, [pi2[m] * f2g[m] for m in range(M)])
            euv = None
            for m1 in range(M):
                mrow = None
                for m2 in range(M):
                    t = pi_s[m1][m2] * f2g[m2]
                    mrow = t if mrow is None else mrow + t
                t = f1g[m1] * mrow
                euv = t if euv is None else euv + t
            fp = eu + ev - euv
            av = a_v[pl.ds(n0 + g * L, L)]
            out_v[pl.ds(g * L, L)] = 1.0 - (1.0 - av) * (1.0 - fp)

        pltpu.sync_copy(out_v, out_hbm.at[pl.ds(n0, NB)])
        return carry

    lax.fori_loop(0, num_j, blk, 0)


@jax.jit
def _run(a, x1r, x2r, w):
    mesh = plsc.VectorSubcoreMesh(
        core_axis_name="c", subcore_axis_name="s",
        num_cores=NC, num_subcores=NS)
    return pl.kernel(
        _tec_body,
        out_type=jax.ShapeDtypeStruct((N,), jnp.float32),
        mesh=mesh,
        compiler_params=pltpu.CompilerParams(use_tc_tiling_on_sc=False, needs_layout_passes=False),
        scratch_types=[
            pltpu.VMEM((N,), jnp.float32),
            pltpu.VMEM((M, NB * PW), jnp.int32),
            pltpu.VMEM((M, NB), jnp.float32),
            pltpu.VMEM((M, NB), jnp.float32),
            pltpu.VMEM((M, M), jnp.float32),
            pltpu.VMEM((NB,), jnp.float32),
        ],
    )(a, x1r, x2r, w)


def kernel(a, X1, X2, W):
    x1r = X1.reshape(M, N * PW)
    x2r = X2.reshape(M, N * PW)
    return _run(a, x1r, x2r, W)
